# Initial kernel scaffold; baseline (speedup 1.0000x reference)
import jax
import jax.numpy as jnp
from jax.experimental import pallas as pl


def kernel(x, tiles, coord, piece, row, col, tilecolor):
    raise NotImplementedError("write your pallas kernel here")


# speedup vs baseline: 9.9823x; 9.9823x over previous
import functools

import jax
import jax.numpy as jnp
from jax import lax
from jax.experimental import pallas as pl
from jax.experimental.pallas import tpu as pltpu
from jax.experimental.pallas import tpu_sc as plsc

DOUT = 64
BATCH = 16384
K = 36
ROWS = 769

NC = 2
NS = 16
NW = NC * NS
BPW = BATCH // NW
CB = 16
NCH = BPW // CB
DPAD = 128


def _table_body(tiles, coord, piece, row, col, tcol, out):
    shp = (12, 8, 8, DOUT)
    i0 = lax.broadcasted_iota(jnp.int32, shp, 0)
    i1 = lax.broadcasted_iota(jnp.int32, shp, 1)
    i2 = lax.broadcasted_iota(jnp.int32, shp, 2)
    special = ((i0 % 6) == 0) & ((i1 == 0) | (i1 == 7))
    white = ((i1 + i2) % 2) == 0
    f = coord[...] + piece[...] + row[...] + col[...] + jnp.where(
        white, tcol[...], jnp.float32(0.0))
    out[...] = jnp.where(special, jnp.float32(0.0), f) + tiles[...]


def _build_table(tiles, coord, piece, row, col, tilecolor):
    w4 = pl.pallas_call(
        _table_body,
        out_shape=jax.ShapeDtypeStruct((12, 8, 8, DOUT), jnp.float32),
    )(tiles, coord, piece, row, col, tilecolor)
    w = w4.reshape(768, DOUT)
    return jnp.zeros((ROWS, DPAD), jnp.float32).at[:768, :DOUT].set(w)


def _emb_body(w_hbm, x_hbm, out_hbm, idx_v, rows_v, out_v, sem):
    wid = lax.axis_index("s") * NC + lax.axis_index("c")
    base = wid * BPW

    def chunk_body(c, carry):
        cb0 = base + c * CB
        pltpu.sync_copy(x_hbm.at[pl.ds(cb0, CB)], idx_v)
        cps = [pltpu.async_copy(w_hbm.at[idx_v.at[b]], rows_v.at[b], sem)
               for b in range(CB)]
        for cp in cps:
            cp.wait()

        def bbody(b, carry2):
            accs = [rows_v[b, 0, pl.ds(16 * j, 16)] for j in range(4)]
            for k in range(1, K):
                for j in range(4):
                    accs[j] = accs[j] + rows_v[b, k, pl.ds(16 * j, 16)]
            for j in range(4):
                out_v[b, pl.ds(16 * j, 16)] = accs[j]
            return carry2

        lax.fori_loop(0, CB, bbody, 0)
        pltpu.sync_copy(out_v, out_hbm.at[pl.ds(cb0, CB)])
        return carry

    lax.fori_loop(0, NCH, chunk_body, 0)


@functools.cache
def _emb_lookup():
    return pl.kernel(
        _emb_body,
        out_type=jax.ShapeDtypeStruct((BATCH, DOUT), jnp.float32),
        mesh=plsc.VectorSubcoreMesh(core_axis_name="c", subcore_axis_name="s"),
        scratch_types=[
            pltpu.VMEM((CB, K), jnp.int32),
            pltpu.VMEM((CB, K, DPAD), jnp.float32),
            pltpu.VMEM((CB, DOUT), jnp.float32),
            pltpu.SemaphoreType.DMA,
        ],
    )


def kernel(x, tiles, coord, piece, row, col, tilecolor):
    w = _build_table(tiles, coord, piece, row, col, tilecolor)
    return _emb_lookup()(w, x.astype(jnp.int32))


# speedup vs baseline: 11.9683x; 1.1990x over previous
import functools

import jax
import jax.numpy as jnp
from jax import lax
from jax.experimental import pallas as pl
from jax.experimental.pallas import tpu as pltpu
from jax.experimental.pallas import tpu_sc as plsc

DOUT = 64
BATCH = 16384
K = 36
ROWS = 769

NC = 2
NS = 16
NW = NC * NS
BPW = BATCH // NW
CB = 32
NCH = BPW // CB
DPAD = 64


def _table_body(tiles, coord, piece, row, col, tcol, out):
    shp = (12, 8, 8, DOUT)
    i0 = lax.broadcasted_iota(jnp.int32, shp, 0)
    i1 = lax.broadcasted_iota(jnp.int32, shp, 1)
    i2 = lax.broadcasted_iota(jnp.int32, shp, 2)
    special = ((i0 % 6) == 0) & ((i1 == 0) | (i1 == 7))
    white = ((i1 + i2) % 2) == 0
    f = coord[...] + piece[...] + row[...] + col[...] + jnp.where(
        white, tcol[...], jnp.float32(0.0))
    out[...] = jnp.where(special, jnp.float32(0.0), f) + tiles[...]


def _build_table(tiles, coord, piece, row, col, tilecolor):
    w4 = pl.pallas_call(
        _table_body,
        out_shape=jax.ShapeDtypeStruct((12, 8, 8, DOUT), jnp.float32),
    )(tiles, coord, piece, row, col, tilecolor)
    w = w4.reshape(768, DOUT)
    return jnp.zeros((ROWS, DPAD), jnp.float32).at[:768, :DOUT].set(w)


def _emb_body(w_hbm, x_hbm, out_hbm, idx_v, rows_v, out_v, sem):
    wid = lax.axis_index("s") * NC + lax.axis_index("c")
    base = wid * BPW

    def chunk_body(c, carry):
        cb0 = base + c * CB
        pltpu.sync_copy(x_hbm.at[pl.ds(cb0, CB)], idx_v)
        cps = [pltpu.async_copy(w_hbm.at[idx_v.at[b]], rows_v.at[b], sem)
               for b in range(CB)]
        for cp in cps:
            cp.wait()

        def bbody(b, carry2):
            accs = [rows_v[b, 0, pl.ds(16 * j, 16)] for j in range(4)]
            for k in range(1, K):
                for j in range(4):
                    accs[j] = accs[j] + rows_v[b, k, pl.ds(16 * j, 16)]
            for j in range(4):
                out_v[b, pl.ds(16 * j, 16)] = accs[j]
            return carry2

        lax.fori_loop(0, CB, bbody, 0)
        pltpu.sync_copy(out_v, out_hbm.at[pl.ds(cb0, CB)])
        return carry

    lax.fori_loop(0, NCH, chunk_body, 0)


@functools.cache
def _emb_lookup():
    return pl.kernel(
        _emb_body,
        out_type=jax.ShapeDtypeStruct((BATCH, DOUT), jnp.float32),
        mesh=plsc.VectorSubcoreMesh(core_axis_name="c", subcore_axis_name="s"),
        compiler_params=pltpu.CompilerParams(use_tc_tiling_on_sc=False),
        scratch_types=[
            pltpu.VMEM((CB, K), jnp.int32),
            pltpu.VMEM((CB, K, DPAD), jnp.float32),
            pltpu.VMEM((CB, DOUT), jnp.float32),
            pltpu.SemaphoreType.DMA,
        ],
    )


def kernel(x, tiles, coord, piece, row, col, tilecolor):
    w = _build_table(tiles, coord, piece, row, col, tilecolor)
    return _emb_lookup()(w, x.astype(jnp.int32))


# speedup vs baseline: 12.2838x; 1.0264x over previous
import functools

import jax
import jax.numpy as jnp
from jax import lax
from jax.experimental import pallas as pl
from jax.experimental.pallas import tpu as pltpu
from jax.experimental.pallas import tpu_sc as plsc

DOUT = 64
BATCH = 16384
K = 36
ROWS = 769

NC = 2
NS = 16
NW = NC * NS
BPW = BATCH // NW
CB = 16
NCH = BPW // CB
DPAD = 64


def _table_body(tiles, coord, piece, row, col, tcol, out):
    shp = (12, 8, 8, DOUT)
    i0 = lax.broadcasted_iota(jnp.int32, shp, 0)
    i1 = lax.broadcasted_iota(jnp.int32, shp, 1)
    i2 = lax.broadcasted_iota(jnp.int32, shp, 2)
    special = ((i0 % 6) == 0) & ((i1 == 0) | (i1 == 7))
    white = ((i1 + i2) % 2) == 0
    f = coord[...] + piece[...] + row[...] + col[...] + jnp.where(
        white, tcol[...], jnp.float32(0.0))
    out[...] = jnp.where(special, jnp.float32(0.0), f) + tiles[...]


def _build_table(tiles, coord, piece, row, col, tilecolor):
    w4 = pl.pallas_call(
        _table_body,
        out_shape=jax.ShapeDtypeStruct((12, 8, 8, DOUT), jnp.float32),
    )(tiles, coord, piece, row, col, tilecolor)
    w = w4.reshape(768, DOUT)
    return jnp.zeros((ROWS, DPAD), jnp.float32).at[:768, :DOUT].set(w)


def _emb_body(w_hbm, x_hbm, out_hbm,
              idx0, idx1, rows0, rows1, out0, out1, sem0, sem1):
    wid = lax.axis_index("s") * NC + lax.axis_index("c")
    base = wid * BPW

    def fire(c, idx_v, rows_v, sem):
        pltpu.sync_copy(x_hbm.at[pl.ds(base + c * CB, CB)], idx_v)
        for b in range(CB):
            pltpu.async_copy(w_hbm.at[idx_v.at[b]],
                             rows_v.at[pl.ds(b * K, K)], sem)

    def drain(rows_v, sem):
        pltpu.make_async_copy(out_hbm.at[pl.ds(0, CB * K)], rows_v, sem).wait()

    def accum(c, rows_v, out_v):
        def bbody(b, carry):
            r0 = b * K
            accs = [rows_v[r0, pl.ds(16 * j, 16)] for j in range(4)]
            for k in range(1, K):
                for j in range(4):
                    accs[j] = accs[j] + rows_v[r0 + k, pl.ds(16 * j, 16)]
            for j in range(4):
                out_v[b, pl.ds(16 * j, 16)] = accs[j]
            return carry

        lax.fori_loop(0, CB, bbody, 0)
        pltpu.sync_copy(out_v, out_hbm.at[pl.ds(base + c * CB, CB)])

    fire(0, idx0, rows0, sem0)

    def step(i, carry):
        c = 2 * i
        fire(c + 1, idx1, rows1, sem1)
        drain(rows0, sem0)
        accum(c, rows0, out0)

        @pl.when(c + 2 < NCH)
        def _():
            fire(c + 2, idx0, rows0, sem0)

        drain(rows1, sem1)
        accum(c + 1, rows1, out1)
        return carry

    lax.fori_loop(0, NCH // 2, step, 0)


@functools.cache
def _emb_lookup():
    return pl.kernel(
        _emb_body,
        out_type=jax.ShapeDtypeStruct((BATCH, DOUT), jnp.float32),
        mesh=plsc.VectorSubcoreMesh(core_axis_name="c", subcore_axis_name="s"),
        compiler_params=pltpu.CompilerParams(use_tc_tiling_on_sc=False),
        scratch_types=[
            pltpu.VMEM((CB, K), jnp.int32),
            pltpu.VMEM((CB, K), jnp.int32),
            pltpu.VMEM((CB * K, DPAD), jnp.float32),
            pltpu.VMEM((CB * K, DPAD), jnp.float32),
            pltpu.VMEM((CB, DOUT), jnp.float32),
            pltpu.VMEM((CB, DOUT), jnp.float32),
            pltpu.SemaphoreType.DMA,
            pltpu.SemaphoreType.DMA,
        ],
    )


def kernel(x, tiles, coord, piece, row, col, tilecolor):
    w = _build_table(tiles, coord, piece, row, col, tilecolor)
    return _emb_lookup()(w, x.astype(jnp.int32))


# speedup vs baseline: 14.3809x; 1.1707x over previous
import functools

import jax
import jax.numpy as jnp
from jax import lax
from jax.experimental import pallas as pl
from jax.experimental.pallas import tpu as pltpu
from jax.experimental.pallas import tpu_sc as plsc

DOUT = 64
BATCH = 16384
K = 36
ROWS = 769

NC = 2
NS = 16
NW = NC * NS
BPW = BATCH // NW
CB = 16
NCH = BPW // CB
DPAD = 64


def _table_body(tiles, coord, piece, row, col, tcol, out):
    shp = (12, 8, 8, DOUT)
    i0 = lax.broadcasted_iota(jnp.int32, shp, 0)
    i1 = lax.broadcasted_iota(jnp.int32, shp, 1)
    i2 = lax.broadcasted_iota(jnp.int32, shp, 2)
    special = ((i0 % 6) == 0) & ((i1 == 0) | (i1 == 7))
    white = ((i1 + i2) % 2) == 0
    f = coord[...] + piece[...] + row[...] + col[...] + jnp.where(
        white, tcol[...], jnp.float32(0.0))
    out[...] = jnp.where(special, jnp.float32(0.0), f) + tiles[...]


def _build_table(tiles, coord, piece, row, col, tilecolor):
    w4 = pl.pallas_call(
        _table_body,
        out_shape=jax.ShapeDtypeStruct((12, 8, 8, DOUT), jnp.float32),
    )(tiles, coord, piece, row, col, tilecolor)
    w = w4.reshape(768, DOUT)
    return jnp.zeros((ROWS, DPAD), jnp.float32).at[:768, :DOUT].set(w)


def _emb_body(w_hbm, x_hbm, out_hbm,
              idx0, idx1, rows0, rows1, out0, out1, sem0, sem1):
    wid = lax.axis_index("s") * NC + lax.axis_index("c")
    base = wid * BPW

    def fire(c, idx_v, rows_v, sem):
        pltpu.sync_copy(x_hbm.at[pl.ds((base + c * CB) * K, CB * K)], idx_v)
        pltpu.async_copy(w_hbm.at[idx_v], rows_v, sem)

    def drain(rows_v, sem):
        pltpu.make_async_copy(out_hbm.at[pl.ds(0, CB * K)], rows_v, sem).wait()

    def accum(c, rows_v, out_v):
        def bbody(b, carry):
            r0 = b * K
            accs = [rows_v[r0, pl.ds(16 * j, 16)] for j in range(4)]
            for k in range(1, K):
                for j in range(4):
                    accs[j] = accs[j] + rows_v[r0 + k, pl.ds(16 * j, 16)]
            for j in range(4):
                out_v[b, pl.ds(16 * j, 16)] = accs[j]
            return carry

        lax.fori_loop(0, CB, bbody, 0)
        pltpu.sync_copy(out_v, out_hbm.at[pl.ds(base + c * CB, CB)])

    fire(0, idx0, rows0, sem0)

    def step(i, carry):
        c = 2 * i
        fire(c + 1, idx1, rows1, sem1)
        drain(rows0, sem0)
        accum(c, rows0, out0)

        @pl.when(c + 2 < NCH)
        def _():
            fire(c + 2, idx0, rows0, sem0)

        drain(rows1, sem1)
        accum(c + 1, rows1, out1)
        return carry

    lax.fori_loop(0, NCH // 2, step, 0)


@functools.cache
def _emb_lookup():
    return pl.kernel(
        _emb_body,
        out_type=jax.ShapeDtypeStruct((BATCH, DOUT), jnp.float32),
        mesh=plsc.VectorSubcoreMesh(core_axis_name="c", subcore_axis_name="s"),
        compiler_params=pltpu.CompilerParams(use_tc_tiling_on_sc=False),
        scratch_types=[
            pltpu.VMEM((CB * K,), jnp.int32),
            pltpu.VMEM((CB * K,), jnp.int32),
            pltpu.VMEM((CB * K, DPAD), jnp.float32),
            pltpu.VMEM((CB * K, DPAD), jnp.float32),
            pltpu.VMEM((CB, DOUT), jnp.float32),
            pltpu.VMEM((CB, DOUT), jnp.float32),
            pltpu.SemaphoreType.DMA,
            pltpu.SemaphoreType.DMA,
        ],
    )


def kernel(x, tiles, coord, piece, row, col, tilecolor):
    w = _build_table(tiles, coord, piece, row, col, tilecolor)
    return _emb_lookup()(w, x.astype(jnp.int32).reshape(-1))
